# baseline (speedup 1.0000x reference)
import functools

import jax
import jax.numpy as jnp
from jax import lax
from jax.experimental import pallas as pl
from jax.experimental.pallas import tpu as pltpu
from jax.experimental.pallas import tpu_sc as plsc

_K = 500
_SLOTS = 4 * _K
_PAD = 2048
_PER_W = _PAD // 16
_VECS = _PER_W // 16
_ROW = 128 * 128


def _sc_body(gidx_hbm, tgtp_hbm, table_hbm, out_hbm,
             gidx_v, tgt_v, vals_v, acc_v, out_v, red_v, shared_s,
             sem, sem2):
    sid = lax.axis_index("s")
    base = sid * _PER_W
    lane = lax.iota(jnp.int32, 16)

    @pl.when(sid == 0)
    def _init():
        red_v[...] = jnp.zeros((16,), jnp.float32)
        pltpu.sync_copy(red_v, shared_s)

    cp_t = pltpu.async_copy(tgtp_hbm.at[pl.ds(base, _PER_W)], tgt_v, sem2)
    pltpu.sync_copy(gidx_hbm.at[pl.ds(base, _PER_W)], gidx_v)
    cp_g = pltpu.async_copy(table_hbm.at[gidx_v], vals_v, sem)
    cp_t.wait()
    cp_g.wait()

    acc = jnp.zeros((16,), jnp.float32)
    for i in range(_VECS):
        v = vals_v[pl.ds(i * 16, 16)]
        t = tgt_v[pl.ds(i * 16, 16)]
        s = base + lane + i * 16
        acc = acc + jnp.where(s < _SLOTS, jnp.abs(v - t), 0.0)
    acc_v[...] = acc

    plsc.subcore_barrier()
    pltpu.sync_copy(acc_v, shared_s.at[lane], add=True)
    plsc.subcore_barrier()

    @pl.when(sid == 0)
    def _reduce():
        pltpu.sync_copy(shared_s, red_v)
        tot = red_v[...]
        t = jnp.float32(0.0)
        for l in range(16):
            t = t + tot[l]
        loss = t * jnp.float32(1.0 / (_K + 0.0001))
        out_v[...] = jnp.where(lane == 0, loss, 0.0)
        pltpu.sync_copy(out_v, out_hbm)


@jax.jit
def kernel(out_vector, target_vector, tgt_indexes):
    b, c, h, w = out_vector.shape
    table = out_vector.reshape(b * c * h * w)
    ind_flat = tgt_indexes.reshape(2 * _K).astype(jnp.int32)
    tgtp = jnp.zeros((_PAD,), jnp.float32).at[:_SLOTS].set(
        target_vector.reshape(_SLOTS))

    s = jnp.arange(_PAD, dtype=jnp.int32)
    d = jnp.bitwise_and(s, 1)
    j = jnp.right_shift(s, 1) % _K
    a = s // (2 * _K)
    g = ind_flat[d * _K + j] + (a * 2 + d) * _ROW
    gidx = jnp.where(s < _SLOTS, g, 0)

    run = functools.partial(
        pl.kernel,
        mesh=plsc.VectorSubcoreMesh(core_axis_name="c", subcore_axis_name="s",
                                    num_cores=1),
        out_type=jax.ShapeDtypeStruct((16,), jnp.float32),
        scratch_types=[
            pltpu.VMEM((_PER_W,), jnp.int32),
            pltpu.VMEM((_PER_W,), jnp.float32),
            pltpu.VMEM((_PER_W,), jnp.float32),
            pltpu.VMEM((16,), jnp.float32),
            pltpu.VMEM((16,), jnp.float32),
            pltpu.VMEM((16,), jnp.float32),
            pltpu.VMEM_SHARED((16,), jnp.float32),
            pltpu.SemaphoreType.DMA,
            pltpu.SemaphoreType.DMA,
        ],
    )(_sc_body)
    out = run(gidx, tgtp, table)
    return out[0]


# speedup vs baseline: 1.5998x; 1.5998x over previous
import functools

import jax
import jax.numpy as jnp
from jax import lax
from jax.experimental import pallas as pl
from jax.experimental.pallas import tpu as pltpu
from jax.experimental.pallas import tpu_sc as plsc

_K = 500
_SLOTS = 4 * _K
_PAD = 2048
_PER_W = _PAD // 16
_VECS = _PER_W // 16
_ROW = 128 * 128


def _sc_body(gidx_hbm, tgtp_hbm, table_hbm, out_hbm,
             gidx_v, tgt_v, vals_v, acc_v, out_v, red_v, shared_s,
             sem, sem2):
    sid = lax.axis_index("s")
    base = sid * _PER_W
    lane = lax.iota(jnp.int32, 16)

    @pl.when(sid == 0)
    def _init():
        red_v[...] = jnp.zeros((16,), jnp.float32)
        pltpu.sync_copy(red_v, shared_s)

    cp_t = pltpu.async_copy(tgtp_hbm.at[pl.ds(base, _PER_W)], tgt_v, sem2)
    pltpu.sync_copy(gidx_hbm.at[pl.ds(base, _PER_W)], gidx_v)
    cp_g = pltpu.async_copy(table_hbm.at[gidx_v], vals_v, sem)
    cp_t.wait()
    cp_g.wait()

    acc = jnp.zeros((16,), jnp.float32)
    for i in range(_VECS):
        v = vals_v[pl.ds(i * 16, 16)]
        t = tgt_v[pl.ds(i * 16, 16)]
        s = base + lane + i * 16
        acc = acc + jnp.where(s < _SLOTS, jnp.abs(v - t), 0.0)
    acc_v[...] = acc

    plsc.subcore_barrier()
    pltpu.sync_copy(acc_v, shared_s.at[lane], add=True)
    plsc.subcore_barrier()

    @pl.when(sid == 0)
    def _reduce():
        pltpu.sync_copy(shared_s, red_v)
        tot = red_v[...]
        t = jnp.float32(0.0)
        for l in range(16):
            t = t + tot[l]
        loss = t * jnp.float32(1.0 / (_K + 0.0001))
        out_v[...] = jnp.where(lane == 0, loss, 0.0)
        pltpu.sync_copy(out_v, out_hbm)


@jax.jit
def kernel(out_vector, target_vector, tgt_indexes):
    b, c, h, w = out_vector.shape
    table = out_vector.reshape(b * c * h * w)
    ind_jd = tgt_indexes.astype(jnp.int32).T.reshape(2 * _K)
    chan_off = jnp.tile(jnp.arange(2, dtype=jnp.int32) * _ROW, _K)
    g_half = ind_jd + chan_off
    gidx = jnp.concatenate(
        [g_half, g_half + 2 * _ROW,
         jnp.zeros((_PAD - _SLOTS,), jnp.int32)])
    tgtp = jnp.concatenate(
        [target_vector.reshape(_SLOTS),
         jnp.zeros((_PAD - _SLOTS,), jnp.float32)])

    run = functools.partial(
        pl.kernel,
        mesh=plsc.VectorSubcoreMesh(core_axis_name="c", subcore_axis_name="s",
                                    num_cores=1),
        out_type=jax.ShapeDtypeStruct((16,), jnp.float32),
        scratch_types=[
            pltpu.VMEM((_PER_W,), jnp.int32),
            pltpu.VMEM((_PER_W,), jnp.float32),
            pltpu.VMEM((_PER_W,), jnp.float32),
            pltpu.VMEM((16,), jnp.float32),
            pltpu.VMEM((16,), jnp.float32),
            pltpu.VMEM((16,), jnp.float32),
            pltpu.VMEM_SHARED((16,), jnp.float32),
            pltpu.SemaphoreType.DMA,
            pltpu.SemaphoreType.DMA,
        ],
    )(_sc_body)
    out = run(gidx, tgtp, table)
    return out[0]
